# Initial kernel scaffold; baseline (speedup 1.0000x reference)
import functools

import jax
import jax.numpy as jnp
from jax import lax
from jax.experimental import pallas as pl
from jax.experimental.pallas import tpu as pltpu
from jax.experimental.pallas import tpu_sc as plsc

NC = 2
NS = 16
NW = NC * NS

DEG_W = 16


def _chunk(per_w):
  for c in range(128, 0, -8):
    if per_w % c == 0:
      return c
  raise ValueError(per_w)


@functools.cache
def _make_deg_kernel(n, per_w, c, steps):
  rows_per_sub = n // NS
  mesh = plsc.VectorSubcoreMesh(core_axis_name="c", subcore_axis_name="s")

  @functools.partial(
      pl.kernel,
      out_type=jax.ShapeDtypeStruct((NC, n, DEG_W), jnp.float32),
      mesh=mesh,
      scratch_types=[
          pltpu.VMEM((steps, c), jnp.int32),
          pltpu.VMEM((c, DEG_W), jnp.float32),
          pltpu.VMEM_SHARED((n, DEG_W), jnp.float32),
      ],
  )
  def deg_kernel(dst_hbm, ones_hbm, zeros_hbm, out_hbm, dst_v, ones_v, acc_sh):
    cid = lax.axis_index("c")
    sid = lax.axis_index("s")
    wid = sid * NC + cid
    r0 = sid * rows_per_sub
    pltpu.sync_copy(zeros_hbm.at[pl.ds(r0, rows_per_sub)],
                    acc_sh.at[pl.ds(r0, rows_per_sub)])
    pltpu.sync_copy(dst_hbm.at[wid], dst_v)
    pltpu.sync_copy(ones_hbm, ones_v)
    plsc.subcore_barrier()

    def step(j, carry):
      pltpu.sync_copy(ones_v, acc_sh.at[dst_v.at[j]], add=True)
      return carry

    lax.fori_loop(0, steps, step, 0)
    plsc.subcore_barrier()
    pltpu.sync_copy(acc_sh.at[pl.ds(r0, rows_per_sub)],
                    out_hbm.at[cid, pl.ds(r0, rows_per_sub)])

  return deg_kernel


@functools.cache
def _make_edge_pass(n, f, per_w, c, steps):
  rows_per_sub = n // NS
  mesh = plsc.VectorSubcoreMesh(core_axis_name="c", subcore_axis_name="s")

  @functools.partial(
      pl.kernel,
      out_type=jax.ShapeDtypeStruct((NC, n, f), jnp.float32),
      mesh=mesh,
      scratch_types=[
          pltpu.VMEM((steps, c), jnp.int32),
          pltpu.VMEM((steps, c), jnp.int32),
          pltpu.VMEM((c, f), jnp.float32),
          pltpu.VMEM_SHARED((n, f), jnp.float32),
          pltpu.SemaphoreType.DMA,
      ],
  )
  def edge_kernel(p_hbm, src_hbm, dst_hbm, zeros_hbm, out_hbm,
                  src_v, dst_v, rows_v, acc_sh, sem):
    cid = lax.axis_index("c")
    sid = lax.axis_index("s")
    wid = sid * NC + cid
    r0 = sid * rows_per_sub
    pltpu.sync_copy(zeros_hbm.at[pl.ds(r0, rows_per_sub)],
                    acc_sh.at[pl.ds(r0, rows_per_sub)])
    pltpu.sync_copy(src_hbm.at[wid], src_v)
    pltpu.sync_copy(dst_hbm.at[wid], dst_v)
    plsc.subcore_barrier()

    def step(j, carry):
      pltpu.async_copy(p_hbm.at[src_v.at[j]], rows_v, sem).wait()
      pltpu.sync_copy(rows_v, acc_sh.at[dst_v.at[j]], add=True)
      return carry

    lax.fori_loop(0, steps, step, 0)
    plsc.subcore_barrier()
    pltpu.sync_copy(acc_sh.at[pl.ds(r0, rows_per_sub)],
                    out_hbm.at[cid, pl.ds(r0, rows_per_sub)])

  return edge_kernel


ROWS = 400


def _g_of(deg_blk):
  return lax.rsqrt(deg_blk[0, :, 0:1] + deg_blk[1, :, 0:1] + 1.0)


def _tc1_body(x_ref, wae_ref, bae_ref, w1_ref, deg_ref, out_ref):
  g = _g_of(deg_ref[...])
  h0 = jnp.dot(x_ref[...], wae_ref[...],
               preferred_element_type=jnp.float32) + bae_ref[...]
  out_ref[...] = g * jnp.dot(h0, w1_ref[...],
                             preferred_element_type=jnp.float32)


def _tc2_body(s1_ref, p1_ref, deg_ref, b1_ref, w2_ref, out_ref):
  g = _g_of(deg_ref[...])
  s = s1_ref[0] + s1_ref[1] + p1_ref[...]
  h1 = jax.nn.sigmoid(g * s + b1_ref[...])
  out_ref[...] = g * jnp.dot(h1, w2_ref[...],
                             preferred_element_type=jnp.float32)


def _tc3_body(s2_ref, p2_ref, deg_ref, b2_ref, out_ref):
  g = _g_of(deg_ref[...])
  s = s2_ref[0] + s2_ref[1] + p2_ref[...]
  out_ref[...] = jax.nn.sigmoid(g * s + b2_ref[...])


def _row_spec(f):
  return pl.BlockSpec((ROWS, f), lambda i: (i, 0))


def _full_spec(a, b):
  return pl.BlockSpec((a, b), lambda i: (0, 0))


def _part_spec(f):
  return pl.BlockSpec((NC, ROWS, f), lambda i: (0, i, 0))


def _deg_spec():
  return pl.BlockSpec((NC, ROWS, DEG_W), lambda i: (0, i, 0))


def kernel(x, edge_index, W_ae, b_ae, W1, b1, W2, b2):
  n, in_dim = x.shape
  hid = W1.shape[0]
  out_dim = W2.shape[1]
  e = edge_index.shape[1]
  per_w = e // NW
  c = _chunk(per_w)
  steps = per_w // c
  grid = (n // ROWS,)

  src_r = edge_index[0].reshape(NW, steps, c)
  dst_r = edge_index[1].reshape(NW, steps, c)

  ones = jnp.ones((c, DEG_W), jnp.float32)
  zeros_deg = jnp.zeros((n, DEG_W), jnp.float32)
  zeros_hid = jnp.zeros((n, hid), jnp.float32)
  zeros_out = jnp.zeros((n, out_dim), jnp.float32)

  deg = _make_deg_kernel(n, per_w, c, steps)(dst_r, ones, zeros_deg)

  p1 = pl.pallas_call(
      _tc1_body,
      grid=grid,
      in_specs=[_row_spec(in_dim), _full_spec(in_dim, hid),
                _full_spec(1, hid), _full_spec(hid, hid), _deg_spec()],
      out_specs=_row_spec(hid),
      out_shape=jax.ShapeDtypeStruct((n, hid), jnp.float32),
  )(x, W_ae, b_ae.reshape(1, hid), W1, deg)

  s1 = _make_edge_pass(n, hid, per_w, c, steps)(p1, src_r, dst_r, zeros_hid)

  p2 = pl.pallas_call(
      _tc2_body,
      grid=grid,
      in_specs=[_part_spec(hid), _row_spec(hid), _deg_spec(),
                _full_spec(1, hid), _full_spec(hid, out_dim)],
      out_specs=_row_spec(out_dim),
      out_shape=jax.ShapeDtypeStruct((n, out_dim), jnp.float32),
  )(s1, p1, deg, b1.reshape(1, hid), W2)

  s2 = _make_edge_pass(n, out_dim, per_w, c, steps)(p2, src_r, dst_r,
                                                    zeros_out)

  out = pl.pallas_call(
      _tc3_body,
      grid=grid,
      in_specs=[_part_spec(out_dim), _row_spec(out_dim), _deg_spec(),
                _full_spec(1, out_dim)],
      out_specs=_row_spec(out_dim),
      out_shape=jax.ShapeDtypeStruct((n, out_dim), jnp.float32),
  )(s2, p2, deg, b2.reshape(1, out_dim))

  return out


# speedup vs baseline: 16.8997x; 16.8997x over previous
import functools

import jax
import jax.numpy as jnp
from jax import lax
from jax.experimental import pallas as pl
from jax.experimental.pallas import tpu as pltpu
from jax.experimental.pallas import tpu_sc as plsc

NC = 2
NS = 16
NW = NC * NS

DEG_W = 128


def _chunk(per_w):
  for c in range(128, 0, -8):
    if per_w % c == 0:
      return c
  raise ValueError(per_w)


def _pad_rows(n):
  per = -(-n // (NS * 8)) * 8
  return NS * per, per


@functools.cache
def _make_deg_kernel(n_pad, per_w, c, steps):
  rows_per_sub = n_pad // NS
  mesh = plsc.VectorSubcoreMesh(core_axis_name="c", subcore_axis_name="s")

  @functools.partial(
      pl.kernel,
      out_type=jax.ShapeDtypeStruct((NC, n_pad, DEG_W), jnp.float32),
      mesh=mesh,
      scratch_types=[
          pltpu.VMEM((steps, c), jnp.int32),
          pltpu.VMEM((c, DEG_W), jnp.float32),
          pltpu.VMEM_SHARED((n_pad, DEG_W), jnp.float32),
      ],
  )
  def deg_kernel(dst_hbm, ones_hbm, zeros_hbm, out_hbm, dst_v, ones_v, acc_sh):
    cid = lax.axis_index("c")
    sid = lax.axis_index("s")
    wid = sid * NC + cid
    r0 = pl.multiple_of(sid * rows_per_sub, 8)
    pltpu.sync_copy(zeros_hbm.at[pl.ds(r0, rows_per_sub)],
                    acc_sh.at[pl.ds(r0, rows_per_sub)])
    pltpu.sync_copy(dst_hbm.at[wid], dst_v)
    pltpu.sync_copy(ones_hbm, ones_v)
    plsc.subcore_barrier()

    def step(j, carry):
      pltpu.sync_copy(ones_v, acc_sh.at[dst_v.at[j]], add=True)
      return carry

    lax.fori_loop(0, steps, step, 0)
    plsc.subcore_barrier()
    pltpu.sync_copy(acc_sh.at[pl.ds(r0, rows_per_sub)],
                    out_hbm.at[cid, pl.ds(r0, rows_per_sub)])

  return deg_kernel


@functools.cache
def _make_edge_pass(n_pad, f, per_w, c, steps):
  rows_per_sub = n_pad // NS
  mesh = plsc.VectorSubcoreMesh(core_axis_name="c", subcore_axis_name="s")

  @functools.partial(
      pl.kernel,
      out_type=jax.ShapeDtypeStruct((NC, n_pad, f), jnp.float32),
      mesh=mesh,
      scratch_types=[
          pltpu.VMEM((steps, c), jnp.int32),
          pltpu.VMEM((steps, c), jnp.int32),
          pltpu.VMEM((c, f), jnp.float32),
          pltpu.VMEM_SHARED((n_pad, f), jnp.float32),
          pltpu.SemaphoreType.DMA,
      ],
  )
  def edge_kernel(p_hbm, src_hbm, dst_hbm, zeros_hbm, out_hbm,
                  src_v, dst_v, rows_v, acc_sh, sem):
    cid = lax.axis_index("c")
    sid = lax.axis_index("s")
    wid = sid * NC + cid
    r0 = pl.multiple_of(sid * rows_per_sub, 8)
    pltpu.sync_copy(zeros_hbm.at[pl.ds(r0, rows_per_sub)],
                    acc_sh.at[pl.ds(r0, rows_per_sub)])
    pltpu.sync_copy(src_hbm.at[wid], src_v)
    pltpu.sync_copy(dst_hbm.at[wid], dst_v)
    plsc.subcore_barrier()

    def step(j, carry):
      pltpu.async_copy(p_hbm.at[src_v.at[j]], rows_v, sem).wait()
      pltpu.sync_copy(rows_v, acc_sh.at[dst_v.at[j]], add=True)
      return carry

    lax.fori_loop(0, steps, step, 0)
    plsc.subcore_barrier()
    pltpu.sync_copy(acc_sh.at[pl.ds(r0, rows_per_sub)],
                    out_hbm.at[cid, pl.ds(r0, rows_per_sub)])

  return edge_kernel


ROWS = 400


def _g_of(deg_blk):
  return lax.rsqrt(deg_blk[0, :, 0:1] + deg_blk[1, :, 0:1] + 1.0)


def _tc1_body(x_ref, wae_ref, bae_ref, w1_ref, deg_ref, out_ref):
  g = _g_of(deg_ref[...])
  h0 = jnp.dot(x_ref[...], wae_ref[...],
               preferred_element_type=jnp.float32) + bae_ref[...]
  out_ref[...] = g * jnp.dot(h0, w1_ref[...],
                             preferred_element_type=jnp.float32)


def _tc2_body(s1_ref, p1_ref, deg_ref, b1_ref, w2_ref, out_ref):
  g = _g_of(deg_ref[...])
  s = s1_ref[0] + s1_ref[1] + p1_ref[...]
  h1 = jax.nn.sigmoid(g * s + b1_ref[...])
  p2 = g * jnp.dot(h1, w2_ref[...], preferred_element_type=jnp.float32)
  out_ref[...] = jnp.concatenate([p2, jnp.zeros_like(p2)], axis=1)


def _tc3_body(s2_ref, p2_ref, deg_ref, b2_ref, out_ref, *, out_dim):
  g = _g_of(deg_ref[...])
  s = (s2_ref[0] + s2_ref[1] + p2_ref[...])[:, :out_dim]
  out_ref[...] = jax.nn.sigmoid(g * s + b2_ref[...])


def _row_spec(f):
  return pl.BlockSpec((ROWS, f), lambda i: (i, 0))


def _full_spec(a, b):
  return pl.BlockSpec((a, b), lambda i: (0, 0))


def _part_spec(f):
  return pl.BlockSpec((NC, ROWS, f), lambda i: (0, i, 0))


def _deg_spec():
  return pl.BlockSpec((NC, ROWS, DEG_W), lambda i: (0, i, 0))


def kernel(x, edge_index, W_ae, b_ae, W1, b1, W2, b2):
  n, in_dim = x.shape
  hid = W1.shape[0]
  out_dim = W2.shape[1]
  e = edge_index.shape[1]
  per_w = e // NW
  c = _chunk(per_w)
  steps = per_w // c
  grid = (n // ROWS,)
  n_pad, _ = _pad_rows(n)

  src_r = edge_index[0].reshape(NW, steps, c)
  dst_r = edge_index[1].reshape(NW, steps, c)

  ones = jnp.ones((c, DEG_W), jnp.float32)
  zeros_hid = jnp.zeros((n_pad, hid), jnp.float32)

  deg = _make_deg_kernel(n_pad, per_w, c, steps)(dst_r, ones, zeros_hid)

  p1 = pl.pallas_call(
      _tc1_body,
      grid=grid,
      in_specs=[_row_spec(in_dim), _full_spec(in_dim, hid),
                _full_spec(1, hid), _full_spec(hid, hid), _deg_spec()],
      out_specs=_row_spec(hid),
      out_shape=jax.ShapeDtypeStruct((n, hid), jnp.float32),
  )(x, W_ae, b_ae.reshape(1, hid), W1, deg)

  s1 = _make_edge_pass(n_pad, hid, per_w, c, steps)(p1, src_r, dst_r,
                                                   zeros_hid)

  p2 = pl.pallas_call(
      _tc2_body,
      grid=grid,
      in_specs=[_part_spec(hid), _row_spec(hid), _deg_spec(),
                _full_spec(1, hid), _full_spec(hid, out_dim)],
      out_specs=_row_spec(2 * out_dim),
      out_shape=jax.ShapeDtypeStruct((n, 2 * out_dim), jnp.float32),
  )(s1, p1, deg, b1.reshape(1, hid), W2)

  s2 = _make_edge_pass(n_pad, 2 * out_dim, per_w, c, steps)(p2, src_r, dst_r,
                                                            zeros_hid)

  out = pl.pallas_call(
      functools.partial(_tc3_body, out_dim=out_dim),
      grid=grid,
      in_specs=[_part_spec(2 * out_dim), _row_spec(2 * out_dim), _deg_spec(),
                _full_spec(1, out_dim)],
      out_specs=_row_spec(out_dim),
      out_shape=jax.ShapeDtypeStruct((n, out_dim), jnp.float32),
  )(s2, p2, deg, b2.reshape(1, out_dim))

  return out
